# Initial kernel scaffold; baseline (speedup 1.0000x reference)
import jax
import jax.numpy as jnp
from jax.experimental import pallas as pl


def kernel(inputs, adj_indices, adj_values, weightAdj_indices, weightAdj_values, featureAdj, W_mlp, b_mlp, W_lp, W_gc1, W_gc2):
    raise NotImplementedError("write your pallas kernel here")


# speedup vs baseline: 2.6506x; 2.6506x over previous
import functools

import jax
import jax.numpy as jnp
from jax import lax
from jax.experimental import pallas as pl
from jax.experimental.pallas import tpu as pltpu
from jax.experimental.pallas import tpu_sc as plsc

N = 10000
E = 160000
D = 256
DH = 128
NC = 2
NS = 16
NW = NC * NS
K = 128
NCHUNK = E // K
NP = 10240
ITERS_ALL = -(-NCHUNK // NW)
ITERS_SUB = -(-NCHUNK // NS)
STRIPE = NP // NS


def _mesh():
    return plsc.VectorSubcoreMesh(
        core_axis_name="c", subcore_axis_name="s", num_cores=NC,
        num_subcores=NS)


def _iota16():
    return lax.iota(jnp.int32, 16)


def _mm_fused_body(x_ref, w_ref, b_ref, lab_ref, whl_ref, whr_ref,
                   s1l_ref, s1r_ref):
    acc = jnp.dot(x_ref[...], w_ref[...], preferred_element_type=jnp.float32)
    lab_ref[...] = acc[:, :D] + b_ref[...][None, :]
    whl_ref[...] = acc[:, D:D + DH]
    whr_ref[...] = acc[:, D + DH:2 * D]
    s1l_ref[...] = acc[:, 2 * D:2 * D + DH]
    s1r_ref[...] = acc[:, 2 * D + DH:3 * D]


def _mm_fused(x, wcat, b):
    blk = 1000
    grid = N // blk
    return pl.pallas_call(
        _mm_fused_body,
        grid=(grid,),
        in_specs=[
            pl.BlockSpec((blk, D), lambda i: (i, 0)),
            pl.BlockSpec((D, 3 * D), lambda i: (0, 0)),
            pl.BlockSpec((D,), lambda i: (0,)),
        ],
        out_specs=[
            pl.BlockSpec((blk, D), lambda i: (i, 0)),
            pl.BlockSpec((blk, DH), lambda i: (i, 0)),
            pl.BlockSpec((blk, DH), lambda i: (i, 0)),
            pl.BlockSpec((blk, DH), lambda i: (i, 0)),
            pl.BlockSpec((blk, DH), lambda i: (i, 0)),
        ],
        out_shape=[
            jax.ShapeDtypeStruct((N, D), jnp.float32),
            jax.ShapeDtypeStruct((N, DH), jnp.float32),
            jax.ShapeDtypeStruct((N, DH), jnp.float32),
            jax.ShapeDtypeStruct((N, DH), jnp.float32),
            jax.ShapeDtypeStruct((N, DH), jnp.float32),
        ],
    )(x, wcat, b)


def _mm2_body(x_ref, w_ref, outl_ref, outr_ref):
    acc = jnp.dot(x_ref[...], w_ref[...], preferred_element_type=jnp.float32)
    outl_ref[...] = acc[:, :DH]
    outr_ref[...] = acc[:, DH:]


def _mm2(x, w):
    blk = 1024
    grid = NP // blk
    return pl.pallas_call(
        _mm2_body,
        grid=(grid,),
        in_specs=[
            pl.BlockSpec((blk, D), lambda i: (i, 0)),
            pl.BlockSpec((D, D), lambda i: (0, 0)),
        ],
        out_specs=[
            pl.BlockSpec((blk, DH), lambda i: (i, 0)),
            pl.BlockSpec((blk, DH), lambda i: (i, 0)),
        ],
        out_shape=[
            jax.ShapeDtypeStruct((NP, DH), jnp.float32),
            jax.ShapeDtypeStruct((NP, DH), jnp.float32),
        ],
    )(x, w)


def _e1_body(lab_hbm, src_hbm, dst_hbm, exp_hbm, rsp_hbm,
             sidx_v, didx_v, rsrc_v, rdst_v, part_v, exp_v, rsl_v, sem):
    c = lax.axis_index("c")
    s = lax.axis_index("s")
    wid = s * NC + c

    def zero_body(r, _):
        rsl_v[pl.ds(r * 16, 16)] = jnp.zeros((16,), jnp.float32)
        return 0
    lax.fori_loop(0, NP // 16, zero_body, 0)

    def chunk(it, _):
        ci = it * NW + wid

        @pl.when(ci < NCHUNK)
        def _():
            base = ci * K
            pltpu.sync_copy(src_hbm.at[pl.ds(base, K)], sidx_v)
            pltpu.sync_copy(dst_hbm.at[pl.ds(base, K)], didx_v)
            d1 = pltpu.async_copy(lab_hbm.at[sidx_v], rsrc_v, sem)
            d2 = pltpu.async_copy(lab_hbm.at[didx_v], rdst_v, sem)
            d1.wait()
            d2.wait()

            def dot_body(e, _):
                acc = rsrc_v[e, pl.ds(0, 16)] * rdst_v[e, pl.ds(0, 16)]
                for j in range(1, 16):
                    acc = acc + (rsrc_v[e, pl.ds(j * 16, 16)] *
                                 rdst_v[e, pl.ds(j * 16, 16)])
                part_v[pl.ds(e * 16, 16)] = acc
                return 0
            lax.fori_loop(0, K, dot_body, 0)

            def red_body(g, _):
                rowid = (_iota16() + g * 16) * 16
                tot = plsc.load_gather(part_v, [rowid])
                for cc in range(1, 16):
                    tot = tot + plsc.load_gather(part_v, [rowid + cc])
                tot = jnp.where(tot > 0, tot, 0.2 * tot)
                ex = jnp.exp(tot)
                exp_v[pl.ds(g * 16, 16)] = ex
                srcv = sidx_v[pl.ds(g * 16, 16)]
                plsc.addupdate_scatter(rsl_v, [srcv], ex)
                return 0
            lax.fori_loop(0, K // 16, red_body, 0)

            pltpu.sync_copy(exp_v, exp_hbm.at[pl.ds(base, K)])
        return 0

    lax.fori_loop(0, ITERS_ALL, chunk, 0)
    pltpu.sync_copy(rsl_v, rsp_hbm.at[pl.ds(wid * NP, NP)])


def _e1(label, src, dst):
    kfn = pl.kernel(
        _e1_body,
        out_type=(
            jax.ShapeDtypeStruct((E,), jnp.float32),
            jax.ShapeDtypeStruct((NW * NP,), jnp.float32),
        ),
        mesh=_mesh(),
        compiler_params=pltpu.CompilerParams(needs_layout_passes=False),
        scratch_types=[
            pltpu.VMEM((K,), jnp.int32),
            pltpu.VMEM((K,), jnp.int32),
            pltpu.VMEM((K, D), jnp.float32),
            pltpu.VMEM((K, D), jnp.float32),
            pltpu.VMEM((K * 16,), jnp.float32),
            pltpu.VMEM((K,), jnp.float32),
            pltpu.VMEM((NP,), jnp.float32),
            pltpu.SemaphoreType.DMA,
        ],
    )
    return kfn(label, src, dst)


def _norm_body(exp_hbm, src_hbm, rsp_hbm, norm_hbm,
               sidx_v, exp_v, norm_v, rs_v, tmp_v, sem):
    c = lax.axis_index("c")
    s = lax.axis_index("s")
    wid = s * NC + c

    pltpu.sync_copy(rsp_hbm.at[pl.ds(0, NP)], rs_v)

    def add_part(p, _):
        pltpu.sync_copy(rsp_hbm.at[pl.ds(p * NP, NP)], tmp_v)

        def add_row(r, _):
            sl = pl.ds(r * 16, 16)
            rs_v[sl] = rs_v[sl] + tmp_v[sl]
            return 0
        lax.fori_loop(0, NP // 16, add_row, 0)
        return 0
    lax.fori_loop(1, NW, add_part, 0)

    def chunk(it, _):
        ci = it * NW + wid

        @pl.when(ci < NCHUNK)
        def _():
            base = ci * K
            pltpu.sync_copy(src_hbm.at[pl.ds(base, K)], sidx_v)
            pltpu.sync_copy(exp_hbm.at[pl.ds(base, K)], exp_v)

            def g_body(g, _):
                sl = pl.ds(g * 16, 16)
                srcv = sidx_v[sl]
                rsv = plsc.load_gather(rs_v, [srcv])
                rsv = jnp.maximum(rsv, 1e-9)
                norm_v[sl] = exp_v[sl] / rsv
                return 0
            lax.fori_loop(0, K // 16, g_body, 0)
            pltpu.sync_copy(norm_v, norm_hbm.at[pl.ds(base, K)])
        return 0

    lax.fori_loop(0, ITERS_ALL, chunk, 0)


def _norm(expE, src, rspart):
    kfn = pl.kernel(
        _norm_body,
        out_type=jax.ShapeDtypeStruct((E,), jnp.float32),
        mesh=_mesh(),
        compiler_params=pltpu.CompilerParams(needs_layout_passes=False),
        scratch_types=[
            pltpu.VMEM((K,), jnp.int32),
            pltpu.VMEM((K,), jnp.float32),
            pltpu.VMEM((K,), jnp.float32),
            pltpu.VMEM((NP,), jnp.float32),
            pltpu.VMEM((NP,), jnp.float32),
            pltpu.SemaphoreType.DMA,
        ],
    )
    return kfn(expE, src, rspart)


def _spmm_body(has_init, oidx_hbm, gidx_hbm, vals_hbm, tabl_hbm, tabr_hbm,
               init_hbm, out_hbm, oidx_v, gidx_v, vals_v, rows_v, acc_sh,
               sem):
    c = lax.axis_index("c")
    s = lax.axis_index("s")

    if has_init:
        @pl.when(c == 0)
        def _():
            pltpu.sync_copy(
                init_hbm.at[pl.ds(s * STRIPE, STRIPE), pl.ds(0, DH)],
                acc_sh.at[pl.ds(s * STRIPE, STRIPE)])

        @pl.when(c == 1)
        def _():
            pltpu.sync_copy(
                init_hbm.at[pl.ds(s * STRIPE, STRIPE), pl.ds(DH, DH)],
                acc_sh.at[pl.ds(s * STRIPE, STRIPE)])
    else:
        def zero_body(r, _):
            z = jnp.zeros((16,), jnp.float32)
            for j in range(8):
                rows_v[r, pl.ds(j * 16, 16)] = z
            return 0
        lax.fori_loop(0, K, zero_body, 0)

        def zcopy(r, _):
            pltpu.sync_copy(
                rows_v, acc_sh.at[pl.ds(s * STRIPE + r * K, K)])
            return 0
        lax.fori_loop(0, STRIPE // K, zcopy, 0)
    plsc.subcore_barrier()

    def chunk(it, _):
        ci = it * NS + s

        @pl.when(ci < NCHUNK)
        def _():
            base = ci * K
            pltpu.sync_copy(oidx_hbm.at[pl.ds(base, K)], oidx_v)
            pltpu.sync_copy(gidx_hbm.at[pl.ds(base, K)], gidx_v)
            pltpu.sync_copy(vals_hbm.at[pl.ds(base, K)], vals_v)

            @pl.when(c == 0)
            def _():
                pltpu.async_copy(tabl_hbm.at[gidx_v], rows_v, sem).wait()

            @pl.when(c == 1)
            def _():
                pltpu.async_copy(tabr_hbm.at[gidx_v], rows_v, sem).wait()

            def scale_body(e, _):
                vsplat = plsc.load_gather(
                    vals_v, [jnp.full((16,), e, jnp.int32)])
                for j in range(8):
                    sl = pl.ds(j * 16, 16)
                    rows_v[e, sl] = rows_v[e, sl] * vsplat
                return 0
            lax.fori_loop(0, K, scale_body, 0)

            pltpu.sync_copy(rows_v, acc_sh.at[oidx_v], add=True)
        return 0

    lax.fori_loop(0, ITERS_SUB, chunk, 0)
    plsc.subcore_barrier()

    @pl.when(c == 0)
    def _():
        pltpu.sync_copy(
            acc_sh.at[pl.ds(s * STRIPE, STRIPE)],
            out_hbm.at[pl.ds(s * STRIPE, STRIPE), pl.ds(0, DH)])

    @pl.when(c == 1)
    def _():
        pltpu.sync_copy(
            acc_sh.at[pl.ds(s * STRIPE, STRIPE)],
            out_hbm.at[pl.ds(s * STRIPE, STRIPE), pl.ds(DH, DH)])


def _spmm(oidx, gidx, vals, tabl, tabr, init):
    has_init = init is not None
    if not has_init:
        init = jnp.zeros((8, D), jnp.float32)
    kfn = pl.kernel(
        functools.partial(_spmm_body, has_init),
        out_type=jax.ShapeDtypeStruct((NP, D), jnp.float32),
        mesh=_mesh(),
        compiler_params=pltpu.CompilerParams(needs_layout_passes=False),
        scratch_types=[
            pltpu.VMEM((K,), jnp.int32),
            pltpu.VMEM((K,), jnp.int32),
            pltpu.VMEM((K,), jnp.float32),
            pltpu.VMEM((K, DH), jnp.float32),
            pltpu.VMEM_SHARED((NP, DH), jnp.float32),
            pltpu.SemaphoreType.DMA,
        ],
    )
    return kfn(oidx, gidx, vals, tabl, tabr, init)


def kernel(inputs, adj_indices, adj_values, weightAdj_indices,
           weightAdj_values, featureAdj, W_mlp, b_mlp, W_lp, W_gc1, W_gc2):
    wcat = jnp.concatenate([W_mlp, W_lp, W_gc1], axis=1)
    label, whl, whr, s1l, s1r = _mm_fused(inputs, wcat, b_mlp)

    src = adj_indices[0]
    dst = adj_indices[1]
    wsrc = weightAdj_indices[0]
    wdst = weightAdj_indices[1]

    expE, rspart = _e1(label, src, dst)
    normE = _norm(expE, src, rspart)

    h_prime = _spmm(src, dst, normE, whl, whr, None)
    h2 = _spmm(wsrc, wdst, weightAdj_values, s1l, s1r, None)
    s2l, s2r = _mm2(h2, W_gc2)
    h_pad = _spmm(wsrc, wdst, weightAdj_values, s2l, s2r, h_prime)
    return (h_pad[:N], label)


# speedup vs baseline: 6.0033x; 2.2649x over previous
import functools

import jax
import jax.numpy as jnp
from jax import lax
from jax.experimental import pallas as pl
from jax.experimental.pallas import tpu as pltpu
from jax.experimental.pallas import tpu_sc as plsc

N = 10000
E = 160000
D = 256
DH = 128
NC = 2
NS = 16
NW = NC * NS
K = 128
NCHUNK = E // K
NP = 10240
ITERS_ALL = -(-NCHUNK // NW)
ITERS_SUB = -(-NCHUNK // NS)
STRIPE = NP // NS


def _mesh():
    return plsc.VectorSubcoreMesh(
        core_axis_name="c", subcore_axis_name="s", num_cores=NC,
        num_subcores=NS)


def _iota16():
    return lax.iota(jnp.int32, 16)


def _mm_fused_body(x_ref, w_ref, b_ref, lab_ref, whl_ref, whr_ref,
                   s1l_ref, s1r_ref):
    acc = jnp.dot(x_ref[...], w_ref[...], preferred_element_type=jnp.float32)
    lab_ref[...] = acc[:, :D] + b_ref[...][None, :]
    whl_ref[...] = acc[:, D:D + DH]
    whr_ref[...] = acc[:, D + DH:2 * D]
    s1l_ref[...] = acc[:, 2 * D:2 * D + DH]
    s1r_ref[...] = acc[:, 2 * D + DH:3 * D]


def _mm_fused(x, wcat, b):
    blk = 1000
    grid = N // blk
    return pl.pallas_call(
        _mm_fused_body,
        grid=(grid,),
        in_specs=[
            pl.BlockSpec((blk, D), lambda i: (i, 0)),
            pl.BlockSpec((D, 3 * D), lambda i: (0, 0)),
            pl.BlockSpec((D,), lambda i: (0,)),
        ],
        out_specs=[
            pl.BlockSpec((blk, D), lambda i: (i, 0)),
            pl.BlockSpec((blk, DH), lambda i: (i, 0)),
            pl.BlockSpec((blk, DH), lambda i: (i, 0)),
            pl.BlockSpec((blk, DH), lambda i: (i, 0)),
            pl.BlockSpec((blk, DH), lambda i: (i, 0)),
        ],
        out_shape=[
            jax.ShapeDtypeStruct((N, D), jnp.float32),
            jax.ShapeDtypeStruct((N, DH), jnp.float32),
            jax.ShapeDtypeStruct((N, DH), jnp.float32),
            jax.ShapeDtypeStruct((N, DH), jnp.float32),
            jax.ShapeDtypeStruct((N, DH), jnp.float32),
        ],
    )(x, wcat, b)


def _mm2_body(x_ref, w_ref, outl_ref, outr_ref):
    acc = jnp.dot(x_ref[...], w_ref[...], preferred_element_type=jnp.float32)
    outl_ref[...] = acc[:, :DH]
    outr_ref[...] = acc[:, DH:]


def _mm2(x, w):
    blk = 1024
    grid = NP // blk
    return pl.pallas_call(
        _mm2_body,
        grid=(grid,),
        in_specs=[
            pl.BlockSpec((blk, D), lambda i: (i, 0)),
            pl.BlockSpec((D, D), lambda i: (0, 0)),
        ],
        out_specs=[
            pl.BlockSpec((blk, DH), lambda i: (i, 0)),
            pl.BlockSpec((blk, DH), lambda i: (i, 0)),
        ],
        out_shape=[
            jax.ShapeDtypeStruct((NP, DH), jnp.float32),
            jax.ShapeDtypeStruct((NP, DH), jnp.float32),
        ],
    )(x, w)


def _e1_body(lab_hbm, src_hbm, dst_hbm, exp_hbm, rs2_hbm,
             sidx_v, didx_v, rsrc_v, rdst_v, part_v, exp_v, rsl_v,
             seg_v, rstage_sh, sem):
    c = lax.axis_index("c")
    s = lax.axis_index("s")
    wid = s * NC + c

    def zero_body(r, _):
        rsl_v[pl.ds(r * 16, 16)] = jnp.zeros((16,), jnp.float32)
        return 0
    lax.fori_loop(0, NP // 16, zero_body, 0)

    def chunk(it, _):
        ci = it * NW + wid

        @pl.when(ci < NCHUNK)
        def _():
            base = ci * K
            pltpu.sync_copy(src_hbm.at[pl.ds(base, K)], sidx_v)
            pltpu.sync_copy(dst_hbm.at[pl.ds(base, K)], didx_v)
            d1 = pltpu.async_copy(lab_hbm.at[sidx_v], rsrc_v, sem)
            d2 = pltpu.async_copy(lab_hbm.at[didx_v], rdst_v, sem)
            d1.wait()
            d2.wait()

            def dot_body(e, _):
                acc = rsrc_v[e, pl.ds(0, 16)] * rdst_v[e, pl.ds(0, 16)]
                for j in range(1, 16):
                    acc = acc + (rsrc_v[e, pl.ds(j * 16, 16)] *
                                 rdst_v[e, pl.ds(j * 16, 16)])
                part_v[pl.ds(e * 16, 16)] = acc
                return 0
            lax.fori_loop(0, K, dot_body, 0)

            def red_body(g, _):
                rowid = (_iota16() + g * 16) * 16
                tot = plsc.load_gather(part_v, [rowid])
                for cc in range(1, 16):
                    tot = tot + plsc.load_gather(part_v, [rowid + cc])
                tot = jnp.where(tot > 0, tot, 0.2 * tot)
                ex = jnp.exp(tot)
                exp_v[pl.ds(g * 16, 16)] = ex
                srcv = sidx_v[pl.ds(g * 16, 16)]
                plsc.addupdate_scatter(rsl_v, [srcv], ex)
                return 0
            lax.fori_loop(0, K // 16, red_body, 0)

            pltpu.sync_copy(exp_v, exp_hbm.at[pl.ds(base, K)])
        return 0

    lax.fori_loop(0, ITERS_ALL, chunk, 0)

    plsc.subcore_barrier()
    pltpu.sync_copy(rsl_v, rstage_sh.at[pl.ds(s * NP, NP)])
    plsc.subcore_barrier()
    seg = 640
    descs = []
    for t in range(NS):
        descs.append(pltpu.async_copy(
            rstage_sh.at[pl.ds(t * NP + s * seg, seg)], seg_v.at[t], sem))
    for d in descs:
        d.wait()

    def seg_add(i, _):
        sl = pl.ds(i * 16, 16)
        acc = seg_v[0, sl]
        for t in range(1, NS):
            acc = acc + seg_v[t, sl]
        seg_v[0, sl] = acc
        return 0
    lax.fori_loop(0, seg // 16, seg_add, 0)
    pltpu.sync_copy(seg_v.at[0], rs2_hbm.at[pl.ds(c * NP + s * seg, seg)])


def _e1(label, src, dst):
    kfn = pl.kernel(
        _e1_body,
        out_type=(
            jax.ShapeDtypeStruct((E,), jnp.float32),
            jax.ShapeDtypeStruct((NC * NP,), jnp.float32),
        ),
        mesh=_mesh(),
        compiler_params=pltpu.CompilerParams(needs_layout_passes=False),
        scratch_types=[
            pltpu.VMEM((K,), jnp.int32),
            pltpu.VMEM((K,), jnp.int32),
            pltpu.VMEM((K, D), jnp.float32),
            pltpu.VMEM((K, D), jnp.float32),
            pltpu.VMEM((K * 16,), jnp.float32),
            pltpu.VMEM((K,), jnp.float32),
            pltpu.VMEM((NP,), jnp.float32),
            pltpu.VMEM((NS, 640), jnp.float32),
            pltpu.VMEM_SHARED((NS * NP,), jnp.float32),
            pltpu.SemaphoreType.DMA,
        ],
    )
    return kfn(label, src, dst)


def _rs_body(a_ref, o_ref):
    o_ref[...] = a_ref[0] + a_ref[1]


def _rs_tot(rs2):
    rs3 = rs2.reshape(NC, NP // 128, 128)
    out = pl.pallas_call(
        _rs_body,
        out_shape=jax.ShapeDtypeStruct((NP // 128, 128), jnp.float32),
    )(rs3)
    return out.reshape(NP)


SK = 64
SEPT = 9984
SCPT = SEPT // SK
SNX = (E - NS * SEPT) // SK


def _spmm_body(att, has_init, oidx_hbm, gidx_hbm, vals_hbm, tabl_hbm,
               tabr_hbm, rs_hbm, init_hbm, out_hbm,
               gidx_v, vals_v, oidx2d, oidxf_v, rows0, rows1, rows2, rows3,
               norm_v, rs_v, acc_sh,
               semg0, semg1, semg2, semg3, sems0, sems1, sems2, sems3,
               semi0, semi1, semi2, semi3):
    c = lax.axis_index("c")
    s = lax.axis_index("s")
    rows = [rows0, rows1, rows2, rows3]
    semg = [semg0, semg1, semg2, semg3]
    sems = [sems0, sems1, sems2, sems3]
    semi = [semi0, semi1, semi2, semi3]
    base_e = s * SEPT

    if has_init:
        @pl.when(c == 0)
        def _():
            pltpu.sync_copy(
                init_hbm.at[pl.ds(s * STRIPE, STRIPE), pl.ds(0, DH)],
                acc_sh.at[pl.ds(s * STRIPE, STRIPE)])

        @pl.when(c == 1)
        def _():
            pltpu.sync_copy(
                init_hbm.at[pl.ds(s * STRIPE, STRIPE), pl.ds(DH, DH)],
                acc_sh.at[pl.ds(s * STRIPE, STRIPE)])
    else:
        def zero_body(r, _):
            z = jnp.zeros((16,), jnp.float32)
            for j in range(8):
                rows0[r, pl.ds(j * 16, 16)] = z
            return 0
        lax.fori_loop(0, SK, zero_body, 0)

        def zcopy(r, _):
            pltpu.sync_copy(rows0, acc_sh.at[pl.ds(s * STRIPE + r * SK, SK)])
            return 0
        lax.fori_loop(0, STRIPE // SK, zcopy, 0)

    if att:
        pltpu.sync_copy(rs_hbm, rs_v)

    plsc.subcore_barrier()

    def eoff(j):
        return base_e + j * SK

    def fire_idx(k, j, off=None):
        off = eoff(j) if off is None else off
        pltpu.async_copy(oidx_hbm.at[pl.ds(off, SK)],
                         oidx2d.at[j % 8], semi[k])
        pltpu.async_copy(gidx_hbm.at[pl.ds(off, SK)],
                         gidx_v.at[k], semi[k])
        pltpu.async_copy(vals_hbm.at[pl.ds(off, SK)],
                         vals_v.at[pl.ds(k * SK, SK)], semi[k])
        if att:
            pltpu.async_copy(oidx_hbm.at[pl.ds(off, SK)],
                             oidxf_v.at[pl.ds(k * SK, SK)], semi[k])

    def wait_idx(k, j, off=None):
        off = eoff(j) if off is None else off
        pltpu.make_async_copy(oidx_hbm.at[pl.ds(off, SK)],
                              oidx2d.at[j % 8], semi[k]).wait()
        pltpu.make_async_copy(gidx_hbm.at[pl.ds(off, SK)],
                              gidx_v.at[k], semi[k]).wait()
        pltpu.make_async_copy(vals_hbm.at[pl.ds(off, SK)],
                              vals_v.at[pl.ds(k * SK, SK)], semi[k]).wait()
        if att:
            pltpu.make_async_copy(oidx_hbm.at[pl.ds(off, SK)],
                                  oidxf_v.at[pl.ds(k * SK, SK)],
                                  semi[k]).wait()

    def fire_gather(k):
        gsl = gidx_v.at[k]

        @pl.when(c == 0)
        def _():
            pltpu.async_copy(tabl_hbm.at[gsl], rows[k], semg[k])

        @pl.when(c == 1)
        def _():
            pltpu.async_copy(tabr_hbm.at[gsl], rows[k], semg[k])

    def wait_gather(k):
        gsl = gidx_v.at[k]

        @pl.when(c == 0)
        def _():
            pltpu.make_async_copy(tabl_hbm.at[gsl], rows[k], semg[k]).wait()

        @pl.when(c == 1)
        def _():
            pltpu.make_async_copy(tabr_hbm.at[gsl], rows[k], semg[k]).wait()

    def fire_scatter(k, j):
        pltpu.async_copy(rows[k], acc_sh.at[oidx2d.at[j % 8]], sems[k],
                         add=True)

    def wait_scatter(k, j):
        pltpu.make_async_copy(rows[k], acc_sh.at[oidx2d.at[j % 8]],
                              sems[k]).wait()

    def scale(k, r8):
        if att:
            def grp(g, _):
                gsl = pl.ds(g * 16, 16)
                oidx16 = oidxf_v[pl.ds(k * SK + g * 16, 16)]
                rsv = plsc.load_gather(rs_v, [oidx16])
                rsv = jnp.maximum(rsv, 1e-9) * (2.0 ** -64)
                ev = vals_v[pl.ds(k * SK + g * 16, 16)] * (2.0 ** -64)
                norm_v[gsl] = ev / rsv
                return 0
            lax.fori_loop(0, SK // 16, grp, 0)

        def sc_body(e2, _):
            for u in range(2):
                e = e2 * 2 + u
                if att:
                    vs = plsc.load_gather(
                        norm_v, [jnp.full((16,), 0, jnp.int32) + e])
                else:
                    vs = plsc.load_gather(
                        vals_v, [jnp.full((16,), k * SK, jnp.int32) + e])
                for jj in range(8):
                    sl = pl.ds(jj * 16, 16)
                    rows[k][e, sl] = rows[k][e, sl] * vs
            return 0
        lax.fori_loop(0, SK // 2, sc_body, 0)

    for j in range(4):
        fire_idx(j, j)
    wait_idx(0, 0)
    fire_gather(0)
    wait_idx(1, 1)
    fire_gather(1)

    def main_body(i4, _):
        for k in range(4):
            j = 4 * i4 + k

            @pl.when(j >= 2)
            def _():
                wait_scatter((k - 2) % 4, j - 2)

            @pl.when(j + 2 < SCPT)
            def _():
                wait_idx((k + 2) % 4, j + 2)
                fire_gather((k + 2) % 4)
            wait_gather(k)
            scale(k, j % 8)
            fire_scatter(k, j)

            @pl.when(j + 4 < SCPT)
            def _():
                fire_idx(k, j + 4)
        return 0

    lax.fori_loop(0, SCPT // 4, main_body, 0)
    wait_scatter((SCPT - 2) % 4, SCPT - 2)
    wait_scatter((SCPT - 1) % 4, SCPT - 1)

    @pl.when(s < SNX)
    def _():
        xoff = NS * SEPT + s * SK
        fire_idx(0, 0, off=xoff)
        wait_idx(0, 0, off=xoff)
        fire_gather(0)
        wait_gather(0)
        scale(0, 0)
        fire_scatter(0, 0)
        wait_scatter(0, 0)

    plsc.subcore_barrier()

    @pl.when(c == 0)
    def _():
        pltpu.sync_copy(
            acc_sh.at[pl.ds(s * STRIPE, STRIPE)],
            out_hbm.at[pl.ds(s * STRIPE, STRIPE), pl.ds(0, DH)])

    @pl.when(c == 1)
    def _():
        pltpu.sync_copy(
            acc_sh.at[pl.ds(s * STRIPE, STRIPE)],
            out_hbm.at[pl.ds(s * STRIPE, STRIPE), pl.ds(DH, DH)])


def _spmm(oidx, gidx, vals, tabl, tabr, rs, init):
    att = rs is not None
    has_init = init is not None
    if not att:
        rs = jnp.zeros((8,), jnp.float32)
    if not has_init:
        init = jnp.zeros((8, D), jnp.float32)
    rs_words = NP if att else 8
    kfn = pl.kernel(
        functools.partial(_spmm_body, att, has_init),
        out_type=jax.ShapeDtypeStruct((NP, D), jnp.float32),
        mesh=_mesh(),
        compiler_params=pltpu.CompilerParams(needs_layout_passes=False),
        scratch_types=[
            pltpu.VMEM((4, SK), jnp.int32),
            pltpu.VMEM((4 * SK,), jnp.float32),
            pltpu.VMEM((8, SK), jnp.int32),
            pltpu.VMEM((4 * SK,), jnp.int32),
            pltpu.VMEM((SK, DH), jnp.float32),
            pltpu.VMEM((SK, DH), jnp.float32),
            pltpu.VMEM((SK, DH), jnp.float32),
            pltpu.VMEM((SK, DH), jnp.float32),
            pltpu.VMEM((SK,), jnp.float32),
            pltpu.VMEM((rs_words,), jnp.float32),
            pltpu.VMEM_SHARED((NP, DH), jnp.float32),
            pltpu.SemaphoreType.DMA,
            pltpu.SemaphoreType.DMA,
            pltpu.SemaphoreType.DMA,
            pltpu.SemaphoreType.DMA,
            pltpu.SemaphoreType.DMA,
            pltpu.SemaphoreType.DMA,
            pltpu.SemaphoreType.DMA,
            pltpu.SemaphoreType.DMA,
            pltpu.SemaphoreType.DMA,
            pltpu.SemaphoreType.DMA,
            pltpu.SemaphoreType.DMA,
            pltpu.SemaphoreType.DMA,
        ],
    )
    return kfn(oidx, gidx, vals, tabl, tabr, rs, init)


def kernel(inputs, adj_indices, adj_values, weightAdj_indices,
           weightAdj_values, featureAdj, W_mlp, b_mlp, W_lp, W_gc1, W_gc2):
    wcat = jnp.concatenate([W_mlp, W_lp, W_gc1], axis=1)
    label, whl, whr, s1l, s1r = _mm_fused(inputs, wcat, b_mlp)

    src = adj_indices[0]
    dst = adj_indices[1]
    wsrc = weightAdj_indices[0]
    wdst = weightAdj_indices[1]

    expE, rs2 = _e1(label, src, dst)
    rs_tot = _rs_tot(rs2)

    h_prime = _spmm(src, dst, expE, whl, whr, rs_tot, None)
    h2 = _spmm(wsrc, wdst, weightAdj_values, s1l, s1r, None, None)
    s2l, s2r = _mm2(h2, W_gc2)
    h_pad = _spmm(wsrc, wdst, weightAdj_values, s2l, s2r, None, h_prime)
    return (h_pad[:N], label)


def kernel_debug(inputs, adj_indices, adj_values, weightAdj_indices,
                 weightAdj_values, featureAdj, W_mlp, b_mlp, W_lp, W_gc1,
                 W_gc2):
    wcat = jnp.concatenate([W_mlp, W_lp, W_gc1], axis=1)
    label, whl, whr, s1l, s1r = _mm_fused(inputs, wcat, b_mlp)
    src = adj_indices[0]
    dst = adj_indices[1]
    wsrc = weightAdj_indices[0]
    wdst = weightAdj_indices[1]
    expE, rs2 = _e1(label, src, dst)
    rs_tot = _rs_tot(rs2)
    h_prime = _spmm(src, dst, expE, whl, whr, rs_tot, None)
    h2 = _spmm(wsrc, wdst, weightAdj_values, s1l, s1r, None, None)
    s2l, s2r = _mm2(h2, W_gc2)
    h_pad = _spmm(wsrc, wdst, weightAdj_values, s2l, s2r, None, h_prime)
    return (h_pad[:N], label, expE, rs_tot, h_prime, h2)


# speedup vs baseline: 7.1118x; 1.1846x over previous
import functools

import jax
import jax.numpy as jnp
from jax import lax
from jax.experimental import pallas as pl
from jax.experimental.pallas import tpu as pltpu
from jax.experimental.pallas import tpu_sc as plsc

N = 10000
E = 160000
D = 256
DH = 128
NC = 2
NS = 16
NW = NC * NS
K = 128
NCHUNK = E // K
NP = 10240
ITERS_ALL = -(-NCHUNK // NW)
ITERS_SUB = -(-NCHUNK // NS)
STRIPE = NP // NS


def _mesh():
    return plsc.VectorSubcoreMesh(
        core_axis_name="c", subcore_axis_name="s", num_cores=NC,
        num_subcores=NS)


def _iota16():
    return lax.iota(jnp.int32, 16)


def _mm_fused_body(x_ref, w_ref, b_ref, lab_ref, whl_ref, whr_ref,
                   s1l_ref, s1r_ref):
    acc = jnp.dot(x_ref[...], w_ref[...], preferred_element_type=jnp.float32)
    lab_ref[...] = acc[:, :D] + b_ref[...][None, :]
    whl_ref[...] = acc[:, D:D + DH]
    whr_ref[...] = acc[:, D + DH:2 * D]
    s1l_ref[...] = acc[:, 2 * D:2 * D + DH]
    s1r_ref[...] = acc[:, 2 * D + DH:3 * D]


def _mm_fused(x, wcat, b):
    blk = 1000
    grid = N // blk
    return pl.pallas_call(
        _mm_fused_body,
        grid=(grid,),
        in_specs=[
            pl.BlockSpec((blk, D), lambda i: (i, 0)),
            pl.BlockSpec((D, 3 * D), lambda i: (0, 0)),
            pl.BlockSpec((D,), lambda i: (0,)),
        ],
        out_specs=[
            pl.BlockSpec((blk, D), lambda i: (i, 0)),
            pl.BlockSpec((blk, DH), lambda i: (i, 0)),
            pl.BlockSpec((blk, DH), lambda i: (i, 0)),
            pl.BlockSpec((blk, DH), lambda i: (i, 0)),
            pl.BlockSpec((blk, DH), lambda i: (i, 0)),
        ],
        out_shape=[
            jax.ShapeDtypeStruct((N, D), jnp.float32),
            jax.ShapeDtypeStruct((N, DH), jnp.float32),
            jax.ShapeDtypeStruct((N, DH), jnp.float32),
            jax.ShapeDtypeStruct((N, DH), jnp.float32),
            jax.ShapeDtypeStruct((N, DH), jnp.float32),
        ],
    )(x, wcat, b)


def _mm2_body(x_ref, w_ref, outl_ref, outr_ref):
    acc = jnp.dot(x_ref[...], w_ref[...], preferred_element_type=jnp.float32)
    outl_ref[...] = acc[:, :DH]
    outr_ref[...] = acc[:, DH:]


def _mm2(x, w):
    blk = 1024
    grid = NP // blk
    return pl.pallas_call(
        _mm2_body,
        grid=(grid,),
        in_specs=[
            pl.BlockSpec((blk, D), lambda i: (i, 0)),
            pl.BlockSpec((D, D), lambda i: (0, 0)),
        ],
        out_specs=[
            pl.BlockSpec((blk, DH), lambda i: (i, 0)),
            pl.BlockSpec((blk, DH), lambda i: (i, 0)),
        ],
        out_shape=[
            jax.ShapeDtypeStruct((NP, DH), jnp.float32),
            jax.ShapeDtypeStruct((NP, DH), jnp.float32),
        ],
    )(x, w)


EK = 64
E1EPT = 4992
E1CPT = E1EPT // EK
E1NX = (E - NW * E1EPT) // EK


def _e1_body(lab_hbm, src_hbm, dst_hbm, exp_hbm, rs2_hbm,
             sidx_v, didx_v, rsrc0, rsrc1, rsrc2, rdst0, rdst1, rdst2,
             part_v, exp_v, rsl_v, seg_v, rstage_sh,
             semi0, semi1, semi2, semg0, semg1, semg2,
             seme0, seme1, seme2, sem):
    c = lax.axis_index("c")
    s = lax.axis_index("s")
    wid = s * NC + c
    rsrc = [rsrc0, rsrc1, rsrc2]
    rdst = [rdst0, rdst1, rdst2]
    semi = [semi0, semi1, semi2]
    semg = [semg0, semg1, semg2]
    seme = [seme0, seme1, seme2]

    def zero_body(r, _):
        rsl_v[pl.ds(r * 16, 16)] = jnp.zeros((16,), jnp.float32)
        return 0
    lax.fori_loop(0, NP // 16, zero_body, 0)

    def eo(j):
        return wid * E1EPT + j * EK

    def fire_idx(k, j, off=None):
        off = eo(j) if off is None else off
        pltpu.async_copy(src_hbm.at[pl.ds(off, EK)], sidx_v.at[k], semi[k])
        pltpu.async_copy(dst_hbm.at[pl.ds(off, EK)], didx_v.at[k], semi[k])

    def wait_idx(k, j, off=None):
        off = eo(j) if off is None else off
        pltpu.make_async_copy(src_hbm.at[pl.ds(off, EK)], sidx_v.at[k],
                              semi[k]).wait()
        pltpu.make_async_copy(dst_hbm.at[pl.ds(off, EK)], didx_v.at[k],
                              semi[k]).wait()

    def fire_gather(k):
        pltpu.async_copy(lab_hbm.at[sidx_v.at[k]], rsrc[k], semg[k])
        pltpu.async_copy(lab_hbm.at[didx_v.at[k]], rdst[k], semg[k])

    def wait_gather(k):
        pltpu.make_async_copy(lab_hbm.at[sidx_v.at[k]], rsrc[k],
                              semg[k]).wait()
        pltpu.make_async_copy(lab_hbm.at[didx_v.at[k]], rdst[k],
                              semg[k]).wait()

    def compute(k):
        def dot_body(e2, _):
            for u in range(2):
                e = e2 * 2 + u
                acc = rsrc[k][e, pl.ds(0, 16)] * rdst[k][e, pl.ds(0, 16)]
                for j in range(1, 16):
                    acc = acc + (rsrc[k][e, pl.ds(j * 16, 16)] *
                                 rdst[k][e, pl.ds(j * 16, 16)])
                part_v[pl.ds(e * 16, 16)] = acc
            return 0
        lax.fori_loop(0, EK // 2, dot_body, 0)

        for g in range(EK // 16):
            rowid = (_iota16() + g * 16) * 16
            tot = plsc.load_gather(part_v, [rowid])
            for cc in range(1, 16):
                tot = tot + plsc.load_gather(part_v, [rowid + cc])
            tot = jnp.where(tot > 0, tot, 0.2 * tot)
            ex = jnp.exp(tot)
            exp_v[k, pl.ds(g * 16, 16)] = ex
            srcv = sidx_v[k, pl.ds(g * 16, 16)]
            plsc.addupdate_scatter(rsl_v, [srcv], ex)

    for j in range(3):
        fire_idx(j, j)
    wait_idx(0, 0)
    fire_gather(0)
    wait_idx(1, 1)
    fire_gather(1)

    def main_body(i3, _):
        for k in range(3):
            j = 3 * i3 + k

            @pl.when(j >= 3)
            def _():
                pltpu.make_async_copy(
                    exp_v.at[k], exp_hbm.at[pl.ds(eo(j - 3), EK)],
                    seme[k]).wait()

            @pl.when(j + 2 < E1CPT)
            def _():
                wait_idx((k + 2) % 3, j + 2)
                fire_gather((k + 2) % 3)
            wait_gather(k)
            compute(k)
            pltpu.async_copy(exp_v.at[k], exp_hbm.at[pl.ds(eo(j), EK)],
                             seme[k])

            @pl.when(j + 3 < E1CPT)
            def _():
                fire_idx(k, j + 3)
        return 0

    lax.fori_loop(0, E1CPT // 3, main_body, 0)
    for j in range(E1CPT - 3, E1CPT):
        pltpu.make_async_copy(exp_v.at[j % 3],
                              exp_hbm.at[pl.ds(eo(j), EK)],
                              seme[j % 3]).wait()

    @pl.when(wid < E1NX)
    def _():
        xoff = NW * E1EPT + wid * EK
        fire_idx(0, 0, off=xoff)
        wait_idx(0, 0, off=xoff)
        fire_gather(0)
        wait_gather(0)
        compute(0)
        pltpu.sync_copy(exp_v.at[0], exp_hbm.at[pl.ds(xoff, EK)])

    plsc.subcore_barrier()
    pltpu.sync_copy(rsl_v, rstage_sh.at[pl.ds(s * NP, NP)])
    plsc.subcore_barrier()
    seg = 640
    for b in range(4):
        descs = []
        for tt in range(4):
            t = b * 4 + tt
            descs.append(pltpu.async_copy(
                rstage_sh.at[pl.ds(t * NP + s * seg, seg)], seg_v.at[tt],
                sem))
        for dsc in descs:
            dsc.wait()

        def seg_add(i, _):
            sl = pl.ds(i * 16, 16)
            acc = seg_v[0, sl]
            for tt in range(1, 4):
                acc = acc + seg_v[tt, sl]
            if b == 0:
                rsl_v[sl] = acc
            else:
                rsl_v[sl] = rsl_v[sl] + acc
            return 0
        lax.fori_loop(0, seg // 16, seg_add, 0)
    pltpu.sync_copy(rsl_v.at[pl.ds(0, seg)],
                    rs2_hbm.at[pl.ds(c * NP + s * seg, seg)])


def _e1(label, src, dst):
    kfn = pl.kernel(
        _e1_body,
        out_type=(
            jax.ShapeDtypeStruct((E,), jnp.float32),
            jax.ShapeDtypeStruct((NC * NP,), jnp.float32),
        ),
        mesh=_mesh(),
        compiler_params=pltpu.CompilerParams(needs_layout_passes=False),
        scratch_types=[
            pltpu.VMEM((3, EK), jnp.int32),
            pltpu.VMEM((3, EK), jnp.int32),
            pltpu.VMEM((EK, D), jnp.float32),
            pltpu.VMEM((EK, D), jnp.float32),
            pltpu.VMEM((EK, D), jnp.float32),
            pltpu.VMEM((EK, D), jnp.float32),
            pltpu.VMEM((EK, D), jnp.float32),
            pltpu.VMEM((EK, D), jnp.float32),
            pltpu.VMEM((EK * 16,), jnp.float32),
            pltpu.VMEM((3, EK), jnp.float32),
            pltpu.VMEM((NP,), jnp.float32),
            pltpu.VMEM((4, 640), jnp.float32),
            pltpu.VMEM_SHARED((NS * NP,), jnp.float32),
            pltpu.SemaphoreType.DMA,
            pltpu.SemaphoreType.DMA,
            pltpu.SemaphoreType.DMA,
            pltpu.SemaphoreType.DMA,
            pltpu.SemaphoreType.DMA,
            pltpu.SemaphoreType.DMA,
            pltpu.SemaphoreType.DMA,
            pltpu.SemaphoreType.DMA,
            pltpu.SemaphoreType.DMA,
            pltpu.SemaphoreType.DMA,
        ],
    )
    return kfn(label, src, dst)


def _rs_body(a_ref, o_ref):
    o_ref[...] = a_ref[0] + a_ref[1]


def _rs_tot(rs2):
    rs3 = rs2.reshape(NC, NP // 128, 128)
    out = pl.pallas_call(
        _rs_body,
        out_shape=jax.ShapeDtypeStruct((NP // 128, 128), jnp.float32),
    )(rs3)
    return out.reshape(NP)


SK = 64
SEPT = 9984
SCPT = SEPT // SK
SNX = (E - NS * SEPT) // SK


def _spmm_body(att, has_init, oidx_hbm, gidx_hbm, vals_hbm, tabl_hbm,
               tabr_hbm, rs_hbm, init_hbm, out_hbm,
               gidx_v, vals_v, oidx2d, oidxf_v, rows0, rows1, rows2, rows3,
               norm_v, rs_v, acc_sh,
               semg0, semg1, semg2, semg3, sems0, sems1, sems2, sems3,
               semi0, semi1, semi2, semi3):
    c = lax.axis_index("c")
    s = lax.axis_index("s")
    rows = [rows0, rows1, rows2, rows3]
    semg = [semg0, semg1, semg2, semg3]
    sems = [sems0, sems1, sems2, sems3]
    semi = [semi0, semi1, semi2, semi3]
    base_e = s * SEPT

    if has_init:
        @pl.when(c == 0)
        def _():
            pltpu.sync_copy(
                init_hbm.at[pl.ds(s * STRIPE, STRIPE), pl.ds(0, DH)],
                acc_sh.at[pl.ds(s * STRIPE, STRIPE)])

        @pl.when(c == 1)
        def _():
            pltpu.sync_copy(
                init_hbm.at[pl.ds(s * STRIPE, STRIPE), pl.ds(DH, DH)],
                acc_sh.at[pl.ds(s * STRIPE, STRIPE)])
    else:
        def zero_body(r, _):
            z = jnp.zeros((16,), jnp.float32)
            for j in range(8):
                rows0[r, pl.ds(j * 16, 16)] = z
            return 0
        lax.fori_loop(0, SK, zero_body, 0)

        def zcopy(r, _):
            pltpu.sync_copy(rows0, acc_sh.at[pl.ds(s * STRIPE + r * SK, SK)])
            return 0
        lax.fori_loop(0, STRIPE // SK, zcopy, 0)

    if att:
        pltpu.sync_copy(rs_hbm, rs_v)

    plsc.subcore_barrier()

    def eoff(j):
        return base_e + j * SK

    def fire_idx(k, j, off=None):
        off = eoff(j) if off is None else off
        pltpu.async_copy(oidx_hbm.at[pl.ds(off, SK)],
                         oidx2d.at[j % 8], semi[k])
        pltpu.async_copy(gidx_hbm.at[pl.ds(off, SK)],
                         gidx_v.at[k], semi[k])
        pltpu.async_copy(vals_hbm.at[pl.ds(off, SK)],
                         vals_v.at[pl.ds(k * SK, SK)], semi[k])
        if att:
            pltpu.async_copy(oidx_hbm.at[pl.ds(off, SK)],
                             oidxf_v.at[pl.ds(k * SK, SK)], semi[k])

    def wait_idx(k, j, off=None):
        off = eoff(j) if off is None else off
        pltpu.make_async_copy(oidx_hbm.at[pl.ds(off, SK)],
                              oidx2d.at[j % 8], semi[k]).wait()
        pltpu.make_async_copy(gidx_hbm.at[pl.ds(off, SK)],
                              gidx_v.at[k], semi[k]).wait()
        pltpu.make_async_copy(vals_hbm.at[pl.ds(off, SK)],
                              vals_v.at[pl.ds(k * SK, SK)], semi[k]).wait()
        if att:
            pltpu.make_async_copy(oidx_hbm.at[pl.ds(off, SK)],
                                  oidxf_v.at[pl.ds(k * SK, SK)],
                                  semi[k]).wait()

    def fire_gather(k):
        gsl = gidx_v.at[k]

        @pl.when(c == 0)
        def _():
            pltpu.async_copy(tabl_hbm.at[gsl], rows[k], semg[k])

        @pl.when(c == 1)
        def _():
            pltpu.async_copy(tabr_hbm.at[gsl], rows[k], semg[k])

    def wait_gather(k):
        gsl = gidx_v.at[k]

        @pl.when(c == 0)
        def _():
            pltpu.make_async_copy(tabl_hbm.at[gsl], rows[k], semg[k]).wait()

        @pl.when(c == 1)
        def _():
            pltpu.make_async_copy(tabr_hbm.at[gsl], rows[k], semg[k]).wait()

    def fire_scatter(k, j):
        pltpu.async_copy(rows[k], acc_sh.at[oidx2d.at[j % 8]], sems[k],
                         add=True)

    def wait_scatter(k, j):
        pltpu.make_async_copy(rows[k], acc_sh.at[oidx2d.at[j % 8]],
                              sems[k]).wait()

    def scale(k, r8):
        if att:
            def grp(g, _):
                gsl = pl.ds(g * 16, 16)
                oidx16 = oidxf_v[pl.ds(k * SK + g * 16, 16)]
                rsv = plsc.load_gather(rs_v, [oidx16])
                rsv = jnp.maximum(rsv, 1e-9) * (2.0 ** -64)
                ev = vals_v[pl.ds(k * SK + g * 16, 16)] * (2.0 ** -64)
                norm_v[gsl] = ev / rsv
                return 0
            lax.fori_loop(0, SK // 16, grp, 0)

        def sc_body(e2, _):
            for u in range(2):
                e = e2 * 2 + u
                if att:
                    vs = plsc.load_gather(
                        norm_v, [jnp.full((16,), 0, jnp.int32) + e])
                else:
                    vs = plsc.load_gather(
                        vals_v, [jnp.full((16,), k * SK, jnp.int32) + e])
                for jj in range(8):
                    sl = pl.ds(jj * 16, 16)
                    rows[k][e, sl] = rows[k][e, sl] * vs
            return 0
        lax.fori_loop(0, SK // 2, sc_body, 0)

    for j in range(4):
        fire_idx(j, j)
    wait_idx(0, 0)
    fire_gather(0)
    wait_idx(1, 1)
    fire_gather(1)

    def main_body(i4, _):
        for k in range(4):
            j = 4 * i4 + k

            @pl.when(j >= 2)
            def _():
                wait_scatter((k - 2) % 4, j - 2)

            @pl.when(j + 2 < SCPT)
            def _():
                wait_idx((k + 2) % 4, j + 2)
                fire_gather((k + 2) % 4)
            wait_gather(k)
            scale(k, j % 8)
            fire_scatter(k, j)

            @pl.when(j + 4 < SCPT)
            def _():
                fire_idx(k, j + 4)
        return 0

    lax.fori_loop(0, SCPT // 4, main_body, 0)
    wait_scatter((SCPT - 2) % 4, SCPT - 2)
    wait_scatter((SCPT - 1) % 4, SCPT - 1)

    @pl.when(s < SNX)
    def _():
        xoff = NS * SEPT + s * SK
        fire_idx(0, 0, off=xoff)
        wait_idx(0, 0, off=xoff)
        fire_gather(0)
        wait_gather(0)
        scale(0, 0)
        fire_scatter(0, 0)
        wait_scatter(0, 0)

    plsc.subcore_barrier()

    @pl.when(c == 0)
    def _():
        pltpu.sync_copy(
            acc_sh.at[pl.ds(s * STRIPE, STRIPE)],
            out_hbm.at[pl.ds(s * STRIPE, STRIPE), pl.ds(0, DH)])

    @pl.when(c == 1)
    def _():
        pltpu.sync_copy(
            acc_sh.at[pl.ds(s * STRIPE, STRIPE)],
            out_hbm.at[pl.ds(s * STRIPE, STRIPE), pl.ds(DH, DH)])


def _spmm(oidx, gidx, vals, tabl, tabr, rs, init):
    att = rs is not None
    has_init = init is not None
    if not att:
        rs = jnp.zeros((8,), jnp.float32)
    if not has_init:
        init = jnp.zeros((8, D), jnp.float32)
    rs_words = NP if att else 8
    kfn = pl.kernel(
        functools.partial(_spmm_body, att, has_init),
        out_type=jax.ShapeDtypeStruct((NP, D), jnp.float32),
        mesh=_mesh(),
        compiler_params=pltpu.CompilerParams(needs_layout_passes=False),
        scratch_types=[
            pltpu.VMEM((4, SK), jnp.int32),
            pltpu.VMEM((4 * SK,), jnp.float32),
            pltpu.VMEM((8, SK), jnp.int32),
            pltpu.VMEM((4 * SK,), jnp.int32),
            pltpu.VMEM((SK, DH), jnp.float32),
            pltpu.VMEM((SK, DH), jnp.float32),
            pltpu.VMEM((SK, DH), jnp.float32),
            pltpu.VMEM((SK, DH), jnp.float32),
            pltpu.VMEM((SK,), jnp.float32),
            pltpu.VMEM((rs_words,), jnp.float32),
            pltpu.VMEM_SHARED((NP, DH), jnp.float32),
            pltpu.SemaphoreType.DMA,
            pltpu.SemaphoreType.DMA,
            pltpu.SemaphoreType.DMA,
            pltpu.SemaphoreType.DMA,
            pltpu.SemaphoreType.DMA,
            pltpu.SemaphoreType.DMA,
            pltpu.SemaphoreType.DMA,
            pltpu.SemaphoreType.DMA,
            pltpu.SemaphoreType.DMA,
            pltpu.SemaphoreType.DMA,
            pltpu.SemaphoreType.DMA,
            pltpu.SemaphoreType.DMA,
        ],
    )
    return kfn(oidx, gidx, vals, tabl, tabr, rs, init)


def kernel(inputs, adj_indices, adj_values, weightAdj_indices,
           weightAdj_values, featureAdj, W_mlp, b_mlp, W_lp, W_gc1, W_gc2):
    wcat = jnp.concatenate([W_mlp, W_lp, W_gc1], axis=1)
    label, whl, whr, s1l, s1r = _mm_fused(inputs, wcat, b_mlp)

    src = adj_indices[0]
    dst = adj_indices[1]
    wsrc = weightAdj_indices[0]
    wdst = weightAdj_indices[1]

    expE, rs2 = _e1(label, src, dst)
    rs_tot = _rs_tot(rs2)

    h_prime = _spmm(src, dst, expE, whl, whr, rs_tot, None)
    h2 = _spmm(wsrc, wdst, weightAdj_values, s1l, s1r, None, None)
    s2l, s2r = _mm2(h2, W_gc2)
    h_pad = _spmm(wsrc, wdst, weightAdj_values, s2l, s2r, None, h_prime)
    return (h_pad[:N], label)


def kernel_debug(inputs, adj_indices, adj_values, weightAdj_indices,
                 weightAdj_values, featureAdj, W_mlp, b_mlp, W_lp, W_gc1,
                 W_gc2):
    wcat = jnp.concatenate([W_mlp, W_lp, W_gc1], axis=1)
    label, whl, whr, s1l, s1r = _mm_fused(inputs, wcat, b_mlp)
    src = adj_indices[0]
    dst = adj_indices[1]
    wsrc = weightAdj_indices[0]
    wdst = weightAdj_indices[1]
    expE, rs2 = _e1(label, src, dst)
    rs_tot = _rs_tot(rs2)
    h_prime = _spmm(src, dst, expE, whl, whr, rs_tot, None)
    h2 = _spmm(wsrc, wdst, weightAdj_values, s1l, s1r, None, None)
    s2l, s2r = _mm2(h2, W_gc2)
    h_pad = _spmm(wsrc, wdst, weightAdj_values, s2l, s2r, None, h_prime)
    return (h_pad[:N], label, expE, rs_tot, h_prime, h2)


# speedup vs baseline: 7.1225x; 1.0015x over previous
import functools

import jax
import jax.numpy as jnp
from jax import lax
from jax.experimental import pallas as pl
from jax.experimental.pallas import tpu as pltpu
from jax.experimental.pallas import tpu_sc as plsc

N = 10000
E = 160000
D = 256
DH = 128
NC = 2
NS = 16
NW = NC * NS
K = 128
NCHUNK = E // K
NP = 10240
ITERS_ALL = -(-NCHUNK // NW)
ITERS_SUB = -(-NCHUNK // NS)
STRIPE = NP // NS


def _mesh():
    return plsc.VectorSubcoreMesh(
        core_axis_name="c", subcore_axis_name="s", num_cores=NC,
        num_subcores=NS)


def _iota16():
    return lax.iota(jnp.int32, 16)


def _mm_fused_body(x_ref, w_ref, b_ref, lab_ref, whl_ref, whr_ref,
                   s1l_ref, s1r_ref):
    acc = jnp.dot(x_ref[...], w_ref[...], preferred_element_type=jnp.float32)
    lab_ref[...] = acc[:, :D] + b_ref[...][None, :]
    whl_ref[...] = acc[:, D:D + DH]
    whr_ref[...] = acc[:, D + DH:2 * D]
    s1l_ref[...] = acc[:, 2 * D:2 * D + DH]
    s1r_ref[...] = acc[:, 2 * D + DH:3 * D]


def _mm_fused(x, wcat, b):
    blk = 1000
    grid = N // blk
    return pl.pallas_call(
        _mm_fused_body,
        grid=(grid,),
        in_specs=[
            pl.BlockSpec((blk, D), lambda i: (i, 0)),
            pl.BlockSpec((D, 3 * D), lambda i: (0, 0)),
            pl.BlockSpec((D,), lambda i: (0,)),
        ],
        out_specs=[
            pl.BlockSpec((blk, D), lambda i: (i, 0)),
            pl.BlockSpec((blk, DH), lambda i: (i, 0)),
            pl.BlockSpec((blk, DH), lambda i: (i, 0)),
            pl.BlockSpec((blk, DH), lambda i: (i, 0)),
            pl.BlockSpec((blk, DH), lambda i: (i, 0)),
        ],
        out_shape=[
            jax.ShapeDtypeStruct((N, D), jnp.float32),
            jax.ShapeDtypeStruct((N, DH), jnp.float32),
            jax.ShapeDtypeStruct((N, DH), jnp.float32),
            jax.ShapeDtypeStruct((N, DH), jnp.float32),
            jax.ShapeDtypeStruct((N, DH), jnp.float32),
        ],
    )(x, wcat, b)


def _mm2_body(x_ref, w_ref, outl_ref, outr_ref):
    acc = jnp.dot(x_ref[...], w_ref[...], preferred_element_type=jnp.float32)
    outl_ref[...] = acc[:, :DH]
    outr_ref[...] = acc[:, DH:]


def _mm2(x, w):
    blk = 1024
    grid = NP // blk
    return pl.pallas_call(
        _mm2_body,
        grid=(grid,),
        in_specs=[
            pl.BlockSpec((blk, D), lambda i: (i, 0)),
            pl.BlockSpec((D, D), lambda i: (0, 0)),
        ],
        out_specs=[
            pl.BlockSpec((blk, DH), lambda i: (i, 0)),
            pl.BlockSpec((blk, DH), lambda i: (i, 0)),
        ],
        out_shape=[
            jax.ShapeDtypeStruct((NP, DH), jnp.float32),
            jax.ShapeDtypeStruct((NP, DH), jnp.float32),
        ],
    )(x, w)


EK = 64
E1EPT = 4992
E1CPT = E1EPT // EK
E1NX = (E - NW * E1EPT) // EK


def _e1_body(lab_hbm, src_hbm, dst_hbm, exp_hbm, rs2_hbm,
             sidx_v, didx_v, rsrc0, rsrc1, rsrc2, rdst0, rdst1, rdst2,
             part_v, exp_v, rsl_v, seg_v, rstage_sh,
             semi0, semi1, semi2, semg0, semg1, semg2,
             seme0, seme1, seme2, sem):
    c = lax.axis_index("c")
    s = lax.axis_index("s")
    wid = s * NC + c
    rsrc = [rsrc0, rsrc1, rsrc2]
    rdst = [rdst0, rdst1, rdst2]
    semi = [semi0, semi1, semi2]
    semg = [semg0, semg1, semg2]
    seme = [seme0, seme1, seme2]

    def zero_body(r, _):
        rsl_v[pl.ds(r * 16, 16)] = jnp.zeros((16,), jnp.float32)
        return 0
    lax.fori_loop(0, NP // 16, zero_body, 0)

    def eo(j):
        return wid * E1EPT + j * EK

    def fire_idx(k, j, off=None):
        off = eo(j) if off is None else off
        pltpu.async_copy(src_hbm.at[pl.ds(off, EK)], sidx_v.at[k], semi[k])
        pltpu.async_copy(dst_hbm.at[pl.ds(off, EK)], didx_v.at[k], semi[k])

    def wait_idx(k, j, off=None):
        off = eo(j) if off is None else off
        pltpu.make_async_copy(src_hbm.at[pl.ds(off, EK)], sidx_v.at[k],
                              semi[k]).wait()
        pltpu.make_async_copy(dst_hbm.at[pl.ds(off, EK)], didx_v.at[k],
                              semi[k]).wait()

    def fire_gather(k):
        pltpu.async_copy(lab_hbm.at[sidx_v.at[k]], rsrc[k], semg[k])
        pltpu.async_copy(lab_hbm.at[didx_v.at[k]], rdst[k], semg[k])

    def wait_gather(k):
        pltpu.make_async_copy(lab_hbm.at[sidx_v.at[k]], rsrc[k],
                              semg[k]).wait()
        pltpu.make_async_copy(lab_hbm.at[didx_v.at[k]], rdst[k],
                              semg[k]).wait()

    def compute(k):
        def dot_body(e2, _):
            for u in range(2):
                e = e2 * 2 + u
                acc = rsrc[k][e, pl.ds(0, 16)] * rdst[k][e, pl.ds(0, 16)]
                for j in range(1, 16):
                    acc = acc + (rsrc[k][e, pl.ds(j * 16, 16)] *
                                 rdst[k][e, pl.ds(j * 16, 16)])
                part_v[pl.ds(e * 16, 16)] = acc
            return 0
        lax.fori_loop(0, EK // 2, dot_body, 0)

        for g in range(EK // 16):
            rowid = (_iota16() + g * 16) * 16
            tot = plsc.load_gather(part_v, [rowid])
            for cc in range(1, 16):
                tot = tot + plsc.load_gather(part_v, [rowid + cc])
            tot = jnp.where(tot > 0, tot, 0.2 * tot)
            ex = jnp.exp(tot)
            exp_v[k, pl.ds(g * 16, 16)] = ex
            srcv = sidx_v[k, pl.ds(g * 16, 16)]
            plsc.addupdate_scatter(rsl_v, [srcv], ex)

    for j in range(3):
        fire_idx(j, j)
    wait_idx(0, 0)
    fire_gather(0)
    wait_idx(1, 1)
    fire_gather(1)

    def main_body(i3, _):
        for k in range(3):
            j = 3 * i3 + k

            @pl.when(j >= 3)
            def _():
                pltpu.make_async_copy(
                    exp_v.at[k], exp_hbm.at[pl.ds(eo(j - 3), EK)],
                    seme[k]).wait()

            @pl.when(j + 2 < E1CPT)
            def _():
                wait_idx((k + 2) % 3, j + 2)
                fire_gather((k + 2) % 3)
            wait_gather(k)
            compute(k)
            pltpu.async_copy(exp_v.at[k], exp_hbm.at[pl.ds(eo(j), EK)],
                             seme[k])

            @pl.when(j + 3 < E1CPT)
            def _():
                fire_idx(k, j + 3)
        return 0

    lax.fori_loop(0, E1CPT // 3, main_body, 0)
    for j in range(E1CPT - 3, E1CPT):
        pltpu.make_async_copy(exp_v.at[j % 3],
                              exp_hbm.at[pl.ds(eo(j), EK)],
                              seme[j % 3]).wait()

    @pl.when(wid < E1NX)
    def _():
        xoff = NW * E1EPT + wid * EK
        fire_idx(0, 0, off=xoff)
        wait_idx(0, 0, off=xoff)
        fire_gather(0)
        wait_gather(0)
        compute(0)
        pltpu.sync_copy(exp_v.at[0], exp_hbm.at[pl.ds(xoff, EK)])

    plsc.subcore_barrier()
    pltpu.sync_copy(rsl_v, rstage_sh.at[pl.ds(s * NP, NP)])
    plsc.subcore_barrier()
    seg = 640
    for b in range(4):
        descs = []
        for tt in range(4):
            t = b * 4 + tt
            descs.append(pltpu.async_copy(
                rstage_sh.at[pl.ds(t * NP + s * seg, seg)], seg_v.at[tt],
                sem))
        for dsc in descs:
            dsc.wait()

        def seg_add(i, _):
            sl = pl.ds(i * 16, 16)
            acc = seg_v[0, sl]
            for tt in range(1, 4):
                acc = acc + seg_v[tt, sl]
            if b == 0:
                rsl_v[sl] = acc
            else:
                rsl_v[sl] = rsl_v[sl] + acc
            return 0
        lax.fori_loop(0, seg // 16, seg_add, 0)
    pltpu.sync_copy(rsl_v.at[pl.ds(0, seg)],
                    rs2_hbm.at[pl.ds(c * NP + s * seg, seg)])


def _e1(label, src, dst):
    kfn = pl.kernel(
        _e1_body,
        out_type=(
            jax.ShapeDtypeStruct((E,), jnp.float32),
            jax.ShapeDtypeStruct((NC * NP,), jnp.float32),
        ),
        mesh=_mesh(),
        compiler_params=pltpu.CompilerParams(needs_layout_passes=False),
        scratch_types=[
            pltpu.VMEM((3, EK), jnp.int32),
            pltpu.VMEM((3, EK), jnp.int32),
            pltpu.VMEM((EK, D), jnp.float32),
            pltpu.VMEM((EK, D), jnp.float32),
            pltpu.VMEM((EK, D), jnp.float32),
            pltpu.VMEM((EK, D), jnp.float32),
            pltpu.VMEM((EK, D), jnp.float32),
            pltpu.VMEM((EK, D), jnp.float32),
            pltpu.VMEM((EK * 16,), jnp.float32),
            pltpu.VMEM((3, EK), jnp.float32),
            pltpu.VMEM((NP,), jnp.float32),
            pltpu.VMEM((4, 640), jnp.float32),
            pltpu.VMEM_SHARED((NS * NP,), jnp.float32),
            pltpu.SemaphoreType.DMA,
            pltpu.SemaphoreType.DMA,
            pltpu.SemaphoreType.DMA,
            pltpu.SemaphoreType.DMA,
            pltpu.SemaphoreType.DMA,
            pltpu.SemaphoreType.DMA,
            pltpu.SemaphoreType.DMA,
            pltpu.SemaphoreType.DMA,
            pltpu.SemaphoreType.DMA,
            pltpu.SemaphoreType.DMA,
        ],
    )
    return kfn(label, src, dst)


def _rs_body(a_ref, o_ref):
    o_ref[...] = a_ref[0] + a_ref[1]


def _rs_tot(rs2):
    rs3 = rs2.reshape(NC, NP // 128, 128)
    out = pl.pallas_call(
        _rs_body,
        out_shape=jax.ShapeDtypeStruct((NP // 128, 128), jnp.float32),
    )(rs3)
    return out.reshape(NP)


SK = 64
SEPT = 9984
SCPT = SEPT // SK
SNX = (E - NS * SEPT) // SK


def _spmm_body(att, has_init, oidx_hbm, gidx_hbm, vals_hbm, tabl_hbm,
               tabr_hbm, rs_hbm, init_hbm, out_hbm,
               gidx_v, vals_v, oidx2d, oidxf_v, rows0, rows1, rows2, rows3,
               norm_v, rs_v, acc_sh,
               semg0, semg1, semg2, semg3, sems0, sems1, sems2, sems3,
               semi0, semi1, semi2, semi3):
    c = lax.axis_index("c")
    s = lax.axis_index("s")
    rows = [rows0, rows1, rows2, rows3]
    semg = [semg0, semg1, semg2, semg3]
    sems = [sems0, sems1, sems2, sems3]
    semi = [semi0, semi1, semi2, semi3]
    base_e = s * SEPT

    if has_init:
        @pl.when(c == 0)
        def _():
            pltpu.sync_copy(
                init_hbm.at[pl.ds(s * STRIPE, STRIPE), pl.ds(0, DH)],
                acc_sh.at[pl.ds(s * STRIPE, STRIPE)])

        @pl.when(c == 1)
        def _():
            pltpu.sync_copy(
                init_hbm.at[pl.ds(s * STRIPE, STRIPE), pl.ds(DH, DH)],
                acc_sh.at[pl.ds(s * STRIPE, STRIPE)])
    else:
        def zero_body(r, _):
            z = jnp.zeros((16,), jnp.float32)
            for j in range(8):
                rows0[r, pl.ds(j * 16, 16)] = z
            return 0
        lax.fori_loop(0, SK, zero_body, 0)

        def zcopy(r, _):
            pltpu.sync_copy(rows0, acc_sh.at[pl.ds(s * STRIPE + r * SK, SK)])
            return 0
        lax.fori_loop(0, STRIPE // SK, zcopy, 0)

    if att:
        pltpu.sync_copy(rs_hbm, rs_v)

    plsc.subcore_barrier()

    def eoff(j):
        return base_e + j * SK

    def fire_idx(k, j, off=None):
        off = eoff(j) if off is None else off
        pltpu.async_copy(oidx_hbm.at[pl.ds(off, SK)],
                         oidx2d.at[j % 8], semi[k])
        pltpu.async_copy(gidx_hbm.at[pl.ds(off, SK)],
                         gidx_v.at[k], semi[k])
        pltpu.async_copy(vals_hbm.at[pl.ds(off, SK)],
                         vals_v.at[pl.ds(k * SK, SK)], semi[k])
        if att:
            pltpu.async_copy(oidx_hbm.at[pl.ds(off, SK)],
                             oidxf_v.at[pl.ds(k * SK, SK)], semi[k])

    def wait_idx(k, j, off=None):
        off = eoff(j) if off is None else off
        pltpu.make_async_copy(oidx_hbm.at[pl.ds(off, SK)],
                              oidx2d.at[j % 8], semi[k]).wait()
        pltpu.make_async_copy(gidx_hbm.at[pl.ds(off, SK)],
                              gidx_v.at[k], semi[k]).wait()
        pltpu.make_async_copy(vals_hbm.at[pl.ds(off, SK)],
                              vals_v.at[pl.ds(k * SK, SK)], semi[k]).wait()
        if att:
            pltpu.make_async_copy(oidx_hbm.at[pl.ds(off, SK)],
                                  oidxf_v.at[pl.ds(k * SK, SK)],
                                  semi[k]).wait()

    def fire_gather(k):
        gsl = gidx_v.at[k]

        @pl.when(c == 0)
        def _():
            pltpu.async_copy(tabl_hbm.at[gsl], rows[k], semg[k])

        @pl.when(c == 1)
        def _():
            pltpu.async_copy(tabr_hbm.at[gsl], rows[k], semg[k])

    def wait_gather(k):
        gsl = gidx_v.at[k]

        @pl.when(c == 0)
        def _():
            pltpu.make_async_copy(tabl_hbm.at[gsl], rows[k], semg[k]).wait()

        @pl.when(c == 1)
        def _():
            pltpu.make_async_copy(tabr_hbm.at[gsl], rows[k], semg[k]).wait()

    def fire_scatter(k, j):
        pltpu.async_copy(rows[k], acc_sh.at[oidx2d.at[j % 8]], sems[k],
                         add=True)

    def wait_scatter(k, j):
        pltpu.make_async_copy(rows[k], acc_sh.at[oidx2d.at[j % 8]],
                              sems[k]).wait()

    def scale(k, r8):
        if att:
            def grp(g, _):
                gsl = pl.ds(g * 16, 16)
                oidx16 = oidxf_v[pl.ds(k * SK + g * 16, 16)]
                rsv = plsc.load_gather(rs_v, [oidx16])
                rsv = jnp.maximum(rsv, 1e-9) * (2.0 ** -64)
                ev = vals_v[pl.ds(k * SK + g * 16, 16)] * (2.0 ** -64)
                norm_v[gsl] = ev / rsv
                return 0
            lax.fori_loop(0, SK // 16, grp, 0)

        def sc_body(e4, _):
            for u in range(4):
                e = e4 * 4 + u
                if att:
                    vs = plsc.load_gather(
                        norm_v, [jnp.full((16,), 0, jnp.int32) + e])
                else:
                    vs = plsc.load_gather(
                        vals_v, [jnp.full((16,), k * SK, jnp.int32) + e])
                for jj in range(8):
                    sl = pl.ds(jj * 16, 16)
                    rows[k][e, sl] = rows[k][e, sl] * vs
            return 0
        lax.fori_loop(0, SK // 4, sc_body, 0)

    for j in range(4):
        fire_idx(j, j)
    wait_idx(0, 0)
    fire_gather(0)
    wait_idx(1, 1)
    fire_gather(1)

    def main_body(i4, _):
        for k in range(4):
            j = 4 * i4 + k

            @pl.when(j >= 2)
            def _():
                wait_scatter((k - 2) % 4, j - 2)

            @pl.when(j + 2 < SCPT)
            def _():
                wait_idx((k + 2) % 4, j + 2)
                fire_gather((k + 2) % 4)
            wait_gather(k)
            scale(k, j % 8)
            fire_scatter(k, j)

            @pl.when(j + 4 < SCPT)
            def _():
                fire_idx(k, j + 4)
        return 0

    lax.fori_loop(0, SCPT // 4, main_body, 0)
    wait_scatter((SCPT - 2) % 4, SCPT - 2)
    wait_scatter((SCPT - 1) % 4, SCPT - 1)

    @pl.when(s < SNX)
    def _():
        xoff = NS * SEPT + s * SK
        fire_idx(0, 0, off=xoff)
        wait_idx(0, 0, off=xoff)
        fire_gather(0)
        wait_gather(0)
        scale(0, 0)
        fire_scatter(0, 0)
        wait_scatter(0, 0)

    plsc.subcore_barrier()

    @pl.when(c == 0)
    def _():
        pltpu.sync_copy(
            acc_sh.at[pl.ds(s * STRIPE, STRIPE)],
            out_hbm.at[pl.ds(s * STRIPE, STRIPE), pl.ds(0, DH)])

    @pl.when(c == 1)
    def _():
        pltpu.sync_copy(
            acc_sh.at[pl.ds(s * STRIPE, STRIPE)],
            out_hbm.at[pl.ds(s * STRIPE, STRIPE), pl.ds(DH, DH)])


def _spmm(oidx, gidx, vals, tabl, tabr, rs, init):
    att = rs is not None
    has_init = init is not None
    if not att:
        rs = jnp.zeros((8,), jnp.float32)
    if not has_init:
        init = jnp.zeros((8, D), jnp.float32)
    rs_words = NP if att else 8
    kfn = pl.kernel(
        functools.partial(_spmm_body, att, has_init),
        out_type=jax.ShapeDtypeStruct((NP, D), jnp.float32),
        mesh=_mesh(),
        compiler_params=pltpu.CompilerParams(needs_layout_passes=False),
        scratch_types=[
            pltpu.VMEM((4, SK), jnp.int32),
            pltpu.VMEM((4 * SK,), jnp.float32),
            pltpu.VMEM((8, SK), jnp.int32),
            pltpu.VMEM((4 * SK,), jnp.int32),
            pltpu.VMEM((SK, DH), jnp.float32),
            pltpu.VMEM((SK, DH), jnp.float32),
            pltpu.VMEM((SK, DH), jnp.float32),
            pltpu.VMEM((SK, DH), jnp.float32),
            pltpu.VMEM((SK,), jnp.float32),
            pltpu.VMEM((rs_words,), jnp.float32),
            pltpu.VMEM_SHARED((NP, DH), jnp.float32),
            pltpu.SemaphoreType.DMA,
            pltpu.SemaphoreType.DMA,
            pltpu.SemaphoreType.DMA,
            pltpu.SemaphoreType.DMA,
            pltpu.SemaphoreType.DMA,
            pltpu.SemaphoreType.DMA,
            pltpu.SemaphoreType.DMA,
            pltpu.SemaphoreType.DMA,
            pltpu.SemaphoreType.DMA,
            pltpu.SemaphoreType.DMA,
            pltpu.SemaphoreType.DMA,
            pltpu.SemaphoreType.DMA,
        ],
    )
    return kfn(oidx, gidx, vals, tabl, tabr, rs, init)


def kernel(inputs, adj_indices, adj_values, weightAdj_indices,
           weightAdj_values, featureAdj, W_mlp, b_mlp, W_lp, W_gc1, W_gc2):
    wcat = jnp.concatenate([W_mlp, W_lp, W_gc1], axis=1)
    label, whl, whr, s1l, s1r = _mm_fused(inputs, wcat, b_mlp)

    src = adj_indices[0]
    dst = adj_indices[1]
    wsrc = weightAdj_indices[0]
    wdst = weightAdj_indices[1]

    expE, rs2 = _e1(label, src, dst)
    rs_tot = _rs_tot(rs2)

    h_prime = _spmm(src, dst, expE, whl, whr, rs_tot, None)
    h2 = _spmm(wsrc, wdst, weightAdj_values, s1l, s1r, None, None)
    s2l, s2r = _mm2(h2, W_gc2)
    h_pad = _spmm(wsrc, wdst, weightAdj_values, s2l, s2r, None, h_prime)
    return (h_pad[:N], label)


def kernel_debug(inputs, adj_indices, adj_values, weightAdj_indices,
                 weightAdj_values, featureAdj, W_mlp, b_mlp, W_lp, W_gc1,
                 W_gc2):
    wcat = jnp.concatenate([W_mlp, W_lp, W_gc1], axis=1)
    label, whl, whr, s1l, s1r = _mm_fused(inputs, wcat, b_mlp)
    src = adj_indices[0]
    dst = adj_indices[1]
    wsrc = weightAdj_indices[0]
    wdst = weightAdj_indices[1]
    expE, rs2 = _e1(label, src, dst)
    rs_tot = _rs_tot(rs2)
    h_prime = _spmm(src, dst, expE, whl, whr, rs_tot, None)
    h2 = _spmm(wsrc, wdst, weightAdj_values, s1l, s1r, None, None)
    s2l, s2r = _mm2(h2, W_gc2)
    h_pad = _spmm(wsrc, wdst, weightAdj_values, s2l, s2r, None, h_prime)
    return (h_pad[:N], label, expE, rs_tot, h_prime, h2)
